# baseline (speedup 1.0000x reference)
import functools

import jax
import jax.numpy as jnp
from jax import lax
from jax.experimental import pallas as pl
from jax.experimental.pallas import tpu as pltpu
from jax.experimental.pallas import tpu_sc as plsc

DIM = 32
B = 4096
LC = 200
LQ = 20

NC = 2
NS = 16
NW = NC * NS

SUB = 128
NBT = B // SUB
NCU = LC // 8 * NBT
NQU = LQ * NBT
CU_W = NCU // NW
QU_W = NQU // NW
NBC = 3 * NBT


def _body(wcV, pcV, kcV, wqV, pqV, kqV,
          W_word, W_pos, W_kg,
          ctxV, qV,
          idx0, idx1, rows0, rows1, stg, sem0, sem1, wsem):
  wid = lax.axis_index("s") * NC + lax.axis_index("c")
  iota = lax.iota(jnp.int32, 16)
  cpat = [(iota + s) % 16 for s in range(16)]
  rpat = [cp // 8 for cp in cpat]
  spat = [(cp % 8) * SUB + iota for cp in cpat]
  idx = (idx0, idx1)
  rows = (rows0, rows1)
  sem = (sem0, sem1)

  def make_job(table, IV, out, t, n_units, nr):
    def fire(u, p):
      g = wid * n_units + u
      pltpu.sync_copy(IV.at[g], idx[p].at[pl.ds(0, nr * SUB)])
      pltpu.async_copy(table.at[idx[p].at[pl.ds(0, nr * SUB)]],
                       rows[p].at[pl.ds(0, nr * SUB)], sem[p])

    def proc(u, p):
      g = wid * n_units + u
      lt = g // NBT
      bt = g % NBT
      pltpu.make_async_copy(table.at[idx[p].at[pl.ds(0, nr * SUB)]],
                            rows[p].at[pl.ds(0, nr * SUB)], sem[p]).wait()

      @pl.loop(0, nr * 8)
      def _(rb, p=p):
        li4 = (rb // 8) * 4
        rx = (rb % 8) * 16
        r0 = rb * 16
        for c in (0, 16):
          for s in range(16):
            v = plsc.load_gather(rows[p], [r0 + iota, cpat[s] + c])
            plsc.store_scatter(stg, [rpat[s] + (li4 + (c // 8)),
                                     spat[s] + rx], v)

      pltpu.async_copy(stg.at[pl.ds(0, 4 * nr)],
                       out.at[lt, pl.ds(0, 4 * nr), NBT * t + bt], wsem)
      pltpu.make_async_copy(stg.at[pl.ds(0, 4 * nr)],
                            out.at[lt, pl.ds(0, 4 * nr), NBT * t + bt],
                            wsem).wait()

    fire(0, 0)

    @pl.loop(0, n_units // 2)
    def _(tt):
      fire(2 * tt + 1, 1)
      proc(2 * tt, 0)

      @pl.when(2 * tt + 2 < n_units)
      def _():
        fire(2 * tt + 2, 0)

      proc(2 * tt + 1, 1)

    if n_units % 2 == 1:
      proc(n_units - 1, 0)

  for t, (table, IV) in enumerate(((W_word, wcV), (W_pos, pcV), (W_kg, kcV))):
    make_job(table, IV, ctxV, t, CU_W, 8)
  for t, (table, QV) in enumerate(((W_word, wqV), (W_pos, pqV), (W_kg, kqV))):
    make_job(table, QV, qV, t, QU_W, 1)


@jax.jit
def _run(wcV, pcV, kcV, wqV, pqV, kqV, W_word, W_pos, W_kg):
  mesh = plsc.VectorSubcoreMesh(core_axis_name="c", subcore_axis_name="s")
  ctxV, qV = pl.kernel(
      _body,
      out_type=(
          jax.ShapeDtypeStruct((LC // 8, 32, NBC, 8 * SUB), jnp.float32),
          jax.ShapeDtypeStruct((LQ, 4, NBC, 8 * SUB), jnp.float32),
      ),
      mesh=mesh,
      compiler_params=pltpu.CompilerParams(use_tc_tiling_on_sc=False,
                                           needs_layout_passes=False),
      scratch_types=[
          pltpu.VMEM((8 * SUB,), jnp.int32),
          pltpu.VMEM((8 * SUB,), jnp.int32),
          pltpu.VMEM((8 * SUB, DIM), jnp.float32),
          pltpu.VMEM((8 * SUB, DIM), jnp.float32),
          pltpu.VMEM((32, 8 * SUB), jnp.float32),
          pltpu.SemaphoreType.DMA,
          pltpu.SemaphoreType.DMA,
          pltpu.SemaphoreType.DMA,
      ],
  )(wcV, pcV, kcV, wqV, pqV, kqV, W_word, W_pos, W_kg)
  return ctxV, qV


def _ctx_idx_view(x):
  return (x.T.reshape(LC // 8, 8, NBT, SUB)
          .transpose(0, 2, 1, 3).reshape(NCU, 8 * SUB))


def _q_idx_view(x):
  return x.T.reshape(NQU, SUB)


def _out_view(y, L):
  return (y.reshape(L, 4, NBC, 8, SUB).transpose(0, 1, 3, 2, 4)
          .reshape(L, DIM, 3 * B).transpose(2, 0, 1))


def kernel(word_context, word_question, kg_context, kg_question,
           pos_context, pos_question, W_word, W_pos, W_kg):
  ctxV, qV = _run(
      _ctx_idx_view(word_context),
      _ctx_idx_view(pos_context),
      _ctx_idx_view(kg_context),
      _q_idx_view(word_question),
      _q_idx_view(pos_question),
      _q_idx_view(kg_question),
      W_word, W_pos, W_kg)
  return (_out_view(ctxV, LC), _out_view(qV, LQ))


# speedup vs baseline: 1.0298x; 1.0298x over previous
import functools

import jax
import jax.numpy as jnp
from jax import lax
from jax.experimental import pallas as pl
from jax.experimental.pallas import tpu as pltpu
from jax.experimental.pallas import tpu_sc as plsc

DIM = 32
B = 4096
LC = 200
LQ = 20

NC = 2
NS = 16
NW = NC * NS

SUB = 128
NBT = B // SUB
NCU = LC // 8 * NBT
NQU = LQ * NBT // 4
CU_W = NCU // NW
QU_W = NQU // NW
NBC = 3 * NBT


def _body(wcV, pcV, kcV, wqV, pqV, kqV,
          W_word, W_pos, W_kg,
          ctxV, qV,
          idx0, idx1, rows0, rows1, stg, sem0, sem1, wsem):
  wid = lax.axis_index("s") * NC + lax.axis_index("c")
  iota = lax.iota(jnp.int32, 16)
  cpat = [(iota + s) % 16 for s in range(16)]
  dpati = [(cp // 8) * 8192 + (cp % 8) * SUB + iota for cp in cpat]
  idx = (idx0, idx1)
  rows = (rows0, rows1)
  sem = (sem0, sem1)

  def make_job(table, IV, out, t, n_units, nbq, lmul):
    def decode(u):
      g = wid * n_units + u
      if nbq == 8:
        return g // 32, (g % 32) // 4, g % 4
      else:
        return g // 8, 0, g % 8

    def fire(u, p):
      lt, li, q = decode(u)
      pltpu.sync_copy(
          IV.at[pl.ds((lt * NBT + q * nbq), nbq), pl.ds(li * SUB, SUB)],
          idx[p].at[pl.ds(0, nbq)])
      for j in range(nbq):
        pltpu.async_copy(table.at[idx[p].at[j]],
                         rows[p].at[pl.ds(j * SUB, SUB)], sem[p])

    def orow(u, kk):
      lt, li, q = decode(u)
      return (lmul * lt + 4 * li + kk) * NBC + NBT * t + q * nbq

    def proc(u, p):
      for j in range(nbq):
        pltpu.make_async_copy(table.at[idx[p].at[j]],
                              rows[p].at[pl.ds(j * SUB, SUB)], sem[p]).wait()

      @pl.loop(0, nbq * 8)
      def _(rb, p=p):
        sb = (rb // 8) * 1024 + (rb % 8) * 16
        r0 = rb * 16
        for c in (0, 16):
          for s in range(16):
            v = plsc.load_gather(rows[p], [r0 + iota, cpat[s] + c])
            plsc.store_scatter(stg, [dpati[s] + (c * 1024 + sb)], v)

      @pl.loop(0, 4)
      def _(kk, u=u):
        pltpu.async_copy(stg.at[pl.ds(kk * 8192, nbq * 1024)],
                         out.at[pl.ds(orow(u, kk) * 1024, nbq * 1024)], wsem)

      @pl.loop(0, 4)
      def _(kk, u=u):
        pltpu.make_async_copy(
            stg.at[pl.ds(kk * 8192, nbq * 1024)],
            out.at[pl.ds(orow(u, kk) * 1024, nbq * 1024)], wsem).wait()

    fire(0, 0)

    @pl.loop(0, n_units // 2)
    def _(tt):
      fire(2 * tt + 1, 1)
      proc(2 * tt, 0)

      @pl.when(2 * tt + 2 < n_units)
      def _():
        fire(2 * tt + 2, 0)

      proc(2 * tt + 1, 1)

    if n_units % 2 == 1:
      proc(n_units - 1, 0)

  for t, (table, IV) in enumerate(((W_word, wcV), (W_pos, pcV), (W_kg, kcV))):
    make_job(table, IV, ctxV, t, CU_W, 8, NBT)
  for t, (table, QV) in enumerate(((W_word, wqV), (W_pos, pqV), (W_kg, kqV))):
    make_job(table, QV, qV, t, QU_W, 4, 4)


@jax.jit
def _run(wcV, pcV, kcV, wqV, pqV, kqV, W_word, W_pos, W_kg):
  mesh = plsc.VectorSubcoreMesh(core_axis_name="c", subcore_axis_name="s")
  ctxV, qV = pl.kernel(
      _body,
      out_type=(
          jax.ShapeDtypeStruct((LC * 4 * NBC * 8 * SUB,), jnp.float32),
          jax.ShapeDtypeStruct((LQ * 4 * NBC * 8 * SUB,), jnp.float32),
      ),
      mesh=mesh,
      compiler_params=pltpu.CompilerParams(use_tc_tiling_on_sc=False,
                                           needs_layout_passes=False),
      scratch_types=[
          pltpu.VMEM((8, SUB), jnp.int32),
          pltpu.VMEM((8, SUB), jnp.int32),
          pltpu.VMEM((8 * SUB, DIM), jnp.float32),
          pltpu.VMEM((8 * SUB, DIM), jnp.float32),
          pltpu.VMEM((4 * 8 * 1024,), jnp.float32),
          pltpu.SemaphoreType.DMA,
          pltpu.SemaphoreType.DMA,
          pltpu.SemaphoreType.DMA,
      ],
  )(wcV, pcV, kcV, wqV, pqV, kqV, W_word, W_pos, W_kg)
  return ctxV, qV


def _ctx_idx_view(x):
  return (x.T.reshape(LC // 8, 8, NBT, SUB)
          .transpose(0, 2, 1, 3).reshape(NCU, 8 * SUB))


def _q_idx_view(x):
  return x.T.reshape(LQ * NBT, SUB)


def _out_view(y, L):
  return (y.reshape(L, 4, NBC, 8, SUB).transpose(0, 1, 3, 2, 4)
          .reshape(L, DIM, 3 * B).transpose(2, 0, 1))


def kernel(word_context, word_question, kg_context, kg_question,
           pos_context, pos_question, W_word, W_pos, W_kg):
  ctxV, qV = _run(
      _ctx_idx_view(word_context),
      _ctx_idx_view(pos_context),
      _ctx_idx_view(kg_context),
      _q_idx_view(word_question),
      _q_idx_view(pos_question),
      _q_idx_view(kg_question),
      W_word, W_pos, W_kg)
  return (_out_view(ctxV, LC), _out_view(qV, LQ))


# speedup vs baseline: 1.4566x; 1.4144x over previous
import functools

import jax
import jax.numpy as jnp
from jax import lax
from jax.experimental import pallas as pl
from jax.experimental.pallas import tpu as pltpu
from jax.experimental.pallas import tpu_sc as plsc

DIM = 32
B = 4096
LC = 200
LQ = 20

NC = 2
NS = 16
NW = NC * NS

SUB = 128
NBT = B // SUB
NCU = LC // 8 * NBT
NQU = LQ * NBT // 4
CU_W = NCU // NW
QU_W = NQU // NW
NBC = 3 * NBT


def _body(wcV, pcV, kcV, wqV, pqV, kqV,
          W_word, W_pos, W_kg,
          ctxV, qV,
          idx0, idx1, rows0, rows1, stg, posv, sem0, sem1, wsem):
  wid = lax.axis_index("s") * NC + lax.axis_index("c")
  iota = lax.iota(jnp.int32, 16)
  cpat = [(iota + s) % 16 for s in range(16)]
  dpati = [(cp // 8) * 8192 + (cp % 8) * SUB + iota for cp in cpat]
  idx = (idx0, idx1)
  rows = (rows0, rows1)
  sem = (sem0, sem1)
  pltpu.sync_copy(W_pos.at[pl.ds(0, 48)], posv.at[pl.ds(0, 48)])
  pltpu.sync_copy(W_pos.at[pl.ds(48, 2)], posv.at[pl.ds(48, 2)])

  def make_job(table, IV, out, t, n_units, nbq, lmul, local=False):
    def decode(u):
      g = wid * n_units + u
      if nbq == 8:
        return g // 32, (g % 32) // 4, g % 4
      else:
        return g // 8, 0, g % 8

    def fire(u, p):
      lt, li, q = decode(u)
      pltpu.sync_copy(
          IV.at[pl.ds((lt * NBT + q * nbq), nbq), pl.ds(li * SUB, SUB)],
          idx[p].at[pl.ds(0, nbq)])
      if not local:
        for j in range(nbq):
          pltpu.async_copy(table.at[idx[p].at[j]],
                           rows[p].at[pl.ds(j * SUB, SUB)], sem[p])

    def orow(u, kk):
      lt, li, q = decode(u)
      return (lmul * lt + 4 * li + kk) * NBC + NBT * t + q * nbq

    def proc(u, p):
      if not local:
        for j in range(nbq):
          pltpu.make_async_copy(table.at[idx[p].at[j]],
                                rows[p].at[pl.ds(j * SUB, SUB)], sem[p]).wait()

      @pl.loop(0, nbq * 8)
      def _(rb, p=p):
        sb = (rb // 8) * 1024 + (rb % 8) * 16
        r0 = rb * 16
        if local:
          ivals = idx[p][rb // 8, pl.ds((rb % 8) * 16, 16)]
        for c in (0, 16):
          for s in range(16):
            if local:
              v = plsc.load_gather(posv, [ivals, cpat[s] + c])
            else:
              v = plsc.load_gather(rows[p], [r0 + iota, cpat[s] + c])
            plsc.store_scatter(stg, [dpati[s] + (c * 1024 + sb)], v)

      @pl.loop(0, 4)
      def _(kk, u=u):
        pltpu.async_copy(stg.at[pl.ds(kk * 8192, nbq * 1024)],
                         out.at[pl.ds(orow(u, kk) * 1024, nbq * 1024)], wsem)

      @pl.loop(0, 4)
      def _(kk, u=u):
        pltpu.make_async_copy(
            stg.at[pl.ds(kk * 8192, nbq * 1024)],
            out.at[pl.ds(orow(u, kk) * 1024, nbq * 1024)], wsem).wait()

    fire(0, 0)

    @pl.loop(0, n_units // 2)
    def _(tt):
      fire(2 * tt + 1, 1)
      proc(2 * tt, 0)

      @pl.when(2 * tt + 2 < n_units)
      def _():
        fire(2 * tt + 2, 0)

      proc(2 * tt + 1, 1)

    if n_units % 2 == 1:
      proc(n_units - 1, 0)

  for t, (table, IV) in enumerate(((W_word, wcV), (W_pos, pcV), (W_kg, kcV))):
    make_job(table, IV, ctxV, t, CU_W, 8, NBT, local=(t == 1))
  for t, (table, QV) in enumerate(((W_word, wqV), (W_pos, pqV), (W_kg, kqV))):
    make_job(table, QV, qV, t, QU_W, 4, 4, local=(t == 1))


@jax.jit
def _run(wcV, pcV, kcV, wqV, pqV, kqV, W_word, W_pos, W_kg):
  mesh = plsc.VectorSubcoreMesh(core_axis_name="c", subcore_axis_name="s")
  ctxV, qV = pl.kernel(
      _body,
      out_type=(
          jax.ShapeDtypeStruct((LC * 4 * NBC * 8 * SUB,), jnp.float32),
          jax.ShapeDtypeStruct((LQ * 4 * NBC * 8 * SUB,), jnp.float32),
      ),
      mesh=mesh,
      compiler_params=pltpu.CompilerParams(use_tc_tiling_on_sc=False,
                                           needs_layout_passes=False),
      scratch_types=[
          pltpu.VMEM((8, SUB), jnp.int32),
          pltpu.VMEM((8, SUB), jnp.int32),
          pltpu.VMEM((8 * SUB, DIM), jnp.float32),
          pltpu.VMEM((8 * SUB, DIM), jnp.float32),
          pltpu.VMEM((4 * 8 * 1024,), jnp.float32),
          pltpu.VMEM((64, DIM), jnp.float32),
          pltpu.SemaphoreType.DMA,
          pltpu.SemaphoreType.DMA,
          pltpu.SemaphoreType.DMA,
      ],
  )(wcV, pcV, kcV, wqV, pqV, kqV, W_word, W_pos, W_kg)
  return ctxV, qV


def _ctx_idx_view(x):
  return (x.T.reshape(LC // 8, 8, NBT, SUB)
          .transpose(0, 2, 1, 3).reshape(NCU, 8 * SUB))


def _q_idx_view(x):
  return x.T.reshape(LQ * NBT, SUB)


def _out_view(y, L):
  return (y.reshape(L, 4, NBC, 8, SUB).transpose(0, 1, 3, 2, 4)
          .reshape(L, DIM, 3 * B).transpose(2, 0, 1))


def kernel(word_context, word_question, kg_context, kg_question,
           pos_context, pos_question, W_word, W_pos, W_kg):
  ctxV, qV = _run(
      _ctx_idx_view(word_context),
      _ctx_idx_view(pos_context),
      _ctx_idx_view(kg_context),
      _q_idx_view(word_question),
      _q_idx_view(pos_question),
      _q_idx_view(kg_question),
      W_word, W_pos, W_kg)
  return (_out_view(ctxV, LC), _out_view(qV, LQ))
